# baseline (speedup 1.0000x reference)
import functools

import jax
import jax.numpy as jnp
from jax import lax
from jax.experimental import pallas as pl
from jax.experimental.pallas import tpu as pltpu
from jax.experimental.pallas import tpu_sc as plsc

VOCAB = 1000
N_EMBD = 32
T = 8
BATCH = 16384
NROW = BATCH * T
NC = 2
NS = 16
NW = NC * NS
PER_W = NROW // NW
C = 128
NCHUNK = PER_W // C

BBLK = 256
NBLK = BATCH // BBLK


@functools.cache
def _make_sc_gather():
    mesh = plsc.VectorSubcoreMesh(core_axis_name="c", subcore_axis_name="s")
    return functools.partial(
        pl.kernel,
        out_type=jax.ShapeDtypeStruct((NROW, N_EMBD), jnp.float32),
        mesh=mesh,
        compiler_params=pltpu.CompilerParams(use_tc_tiling_on_sc=False),
        scratch_types=[
            pltpu.VMEM((NCHUNK, C), jnp.int32),
            pltpu.VMEM((NCHUNK, C), jnp.int32),
            pltpu.VMEM((C, N_EMBD), jnp.float32),
            pltpu.VMEM((C, N_EMBD), jnp.float32),
            pltpu.SemaphoreType.DMA,
            pltpu.SemaphoreType.DMA,
            pltpu.SemaphoreType.DMA,
            pltpu.SemaphoreType.DMA,
        ],
    )(_sc_body)


def _sc_body(idx_hbm, tok_hbm, out_hbm, idx_v, sidx_v,
             buf_a, buf_b, gs_a, gs_b, ss_a, ss_b):
    wid = lax.axis_index("s") * NC + lax.axis_index("c")
    base = wid * PER_W
    pltpu.sync_copy(idx_hbm.at[pl.ds(wid * NCHUNK, NCHUNK), :], idx_v)

    iota = lax.iota(jnp.int32, 16)
    t_vec = iota & 7
    soff = (t_vec >> 2) * (4 * BATCH) + (t_vec & 3)

    def pbody(m, carry):
        j0 = base + m * 16
        b_vec = (j0 >> 3) + (iota >> 3)
        k = m // (C // 16)
        col = (m % (C // 16)) * 16
        sidx_v[k, pl.ds(col, 16)] = soff + b_vec * 4
        return carry

    lax.fori_loop(0, PER_W // 16, pbody, 0)

    bufs = (buf_a, buf_b)
    gsems = (gs_a, gs_b)
    ssems = (ss_a, ss_b)

    def g_start(k, p):
        pltpu.async_copy(tok_hbm.at[idx_v.at[k]], bufs[p], gsems[p])

    def g_wait(k, p):
        pltpu.make_async_copy(tok_hbm.at[idx_v.at[k]], bufs[p], gsems[p]).wait()

    def s_start(k, p):
        pltpu.async_copy(bufs[p], out_hbm.at[sidx_v.at[k]], ssems[p])

    def s_wait(k, p):
        pltpu.make_async_copy(bufs[p], out_hbm.at[sidx_v.at[k]], ssems[p]).wait()

    g_start(0, 0)
    g_start(1, 1)

    def lbody(k2, carry):
        for p in range(2):
            k = k2 * 2 + p
            g_wait(k, p)
            s_start(k, p)
            s_wait(k, p)

            @pl.when(k2 < NCHUNK // 2 - 1)
            def _():
                g_start(k + 2, p)
        return carry

    lax.fori_loop(0, NCHUNK // 2, lbody, 0)


def _proj_body(g_ref, pos_ref, w_ref, b_ref, out_ref):
    for t in range(T):
        h, q = t // 4, t % 4
        x = g_ref[h, :, q * N_EMBD:(q + 1) * N_EMBD] + pos_ref[pl.ds(t, 1), :]
        y = lax.dot_general(w_ref[...], x.astype(jnp.bfloat16),
                            (((1,), (1,)), ((), ())),
                            preferred_element_type=jnp.float32)
        out_ref[t] = y + b_ref[...]


def _project(g2, pos_table, w_bf, b_col):
    return pl.pallas_call(
        _proj_body,
        grid=(NBLK,),
        in_specs=[
            pl.BlockSpec((2, BBLK, 128), lambda k: (0, k, 0)),
            pl.BlockSpec((T, N_EMBD), lambda k: (0, 0)),
            pl.BlockSpec((VOCAB, N_EMBD), lambda k: (0, 0)),
            pl.BlockSpec((VOCAB, 1), lambda k: (0, 0)),
        ],
        out_specs=pl.BlockSpec((T, VOCAB, BBLK), lambda k: (0, 0, k)),
        out_shape=jax.ShapeDtypeStruct((T, VOCAB, BATCH), jnp.float32),
    )(g2, pos_table, w_bf, b_col)


def kernel(idx, tok_table, pos_table, W, b):
    g = _make_sc_gather()(idx.reshape(NROW // C, C), tok_table)
    out3 = _project(g.reshape(2, BATCH, 128), pos_table,
                    W.astype(jnp.bfloat16), b.reshape(VOCAB, 1))
    return jnp.transpose(out3, (2, 0, 1))


# speedup vs baseline: 1.0252x; 1.0252x over previous
import functools

import jax
import jax.numpy as jnp
from jax import lax
from jax.experimental import pallas as pl
from jax.experimental.pallas import tpu as pltpu
from jax.experimental.pallas import tpu_sc as plsc

VOCAB = 1000
N_EMBD = 32
T = 8
BATCH = 16384
NROW = BATCH * T
NC = 2
NS = 16
NW = NC * NS
PER_W = NROW // NW
C = 128
NCHUNK = PER_W // C

BBLK = 1024
NBLK = BATCH // BBLK


@functools.cache
def _make_sc_gather():
    mesh = plsc.VectorSubcoreMesh(core_axis_name="c", subcore_axis_name="s")
    return functools.partial(
        pl.kernel,
        out_type=jax.ShapeDtypeStruct((NROW, N_EMBD), jnp.float32),
        mesh=mesh,
        compiler_params=pltpu.CompilerParams(use_tc_tiling_on_sc=False),
        scratch_types=[
            pltpu.VMEM((NCHUNK, C), jnp.int32),
            pltpu.VMEM((NCHUNK, C), jnp.int32),
            pltpu.VMEM((C, N_EMBD), jnp.float32),
            pltpu.VMEM((C, N_EMBD), jnp.float32),
            pltpu.SemaphoreType.DMA,
            pltpu.SemaphoreType.DMA,
            pltpu.SemaphoreType.DMA,
            pltpu.SemaphoreType.DMA,
        ],
    )(_sc_body)


def _sc_body(idx_hbm, tok_hbm, out_hbm, idx_v, sidx_v,
             buf_a, buf_b, gs_a, gs_b, ss_a, ss_b):
    wid = lax.axis_index("s") * NC + lax.axis_index("c")
    base = wid * PER_W
    pltpu.sync_copy(idx_hbm.at[pl.ds(wid * NCHUNK, NCHUNK), :], idx_v)

    iota = lax.iota(jnp.int32, 16)
    t_vec = iota & 7
    soff = (t_vec >> 2) * (4 * BATCH) + (t_vec & 3)

    def pbody(m, carry):
        j0 = base + m * 16
        b_vec = (j0 >> 3) + (iota >> 3)
        k = m // (C // 16)
        col = (m % (C // 16)) * 16
        sidx_v[k, pl.ds(col, 16)] = soff + b_vec * 4
        return carry

    lax.fori_loop(0, PER_W // 16, pbody, 0)

    bufs = (buf_a, buf_b)
    gsems = (gs_a, gs_b)
    ssems = (ss_a, ss_b)

    def g_start(k, p):
        pltpu.async_copy(tok_hbm.at[idx_v.at[k]], bufs[p], gsems[p])

    def g_wait(k, p):
        pltpu.make_async_copy(tok_hbm.at[idx_v.at[k]], bufs[p], gsems[p]).wait()

    def s_start(k, p):
        pltpu.async_copy(bufs[p], out_hbm.at[sidx_v.at[k]], ssems[p])

    def s_wait(k, p):
        pltpu.make_async_copy(bufs[p], out_hbm.at[sidx_v.at[k]], ssems[p]).wait()

    g_start(0, 0)
    g_start(1, 1)

    def lbody(k2, carry):
        for p in range(2):
            k = k2 * 2 + p
            g_wait(k, p)
            s_start(k, p)
            s_wait(k, p)

            @pl.when(k2 < NCHUNK // 2 - 1)
            def _():
                g_start(k + 2, p)
        return carry

    lax.fori_loop(0, NCHUNK // 2, lbody, 0)


def _proj_body(g_ref, pos_ref, w_ref, b_ref, out_ref):
    hh = pl.program_id(0)
    for q in range(4):
        x = g_ref[0, :, q * N_EMBD:(q + 1) * N_EMBD] \
            + pos_ref[pl.ds(hh * 4 + q, 1), :]
        y = lax.dot_general(w_ref[...], x.astype(jnp.bfloat16),
                            (((1,), (1,)), ((), ())),
                            preferred_element_type=jnp.float32)
        out_ref[q] = y + b_ref[...]


def _project(g2, pos_table, w_bf, b_col):
    return pl.pallas_call(
        _proj_body,
        grid=(2, NBLK),
        in_specs=[
            pl.BlockSpec((1, BBLK, 128), lambda h, k: (h, k, 0)),
            pl.BlockSpec((T, N_EMBD), lambda h, k: (0, 0)),
            pl.BlockSpec((VOCAB, N_EMBD), lambda h, k: (0, 0)),
            pl.BlockSpec((VOCAB, 1), lambda h, k: (0, 0)),
        ],
        out_specs=pl.BlockSpec((4, VOCAB, BBLK), lambda h, k: (h, 0, k)),
        out_shape=jax.ShapeDtypeStruct((T, VOCAB, BATCH), jnp.float32),
    )(g2, pos_table, w_bf, b_col)


def kernel(idx, tok_table, pos_table, W, b):
    g = _make_sc_gather()(idx.reshape(NROW // C, C), tok_table)
    out3 = _project(g.reshape(2, BATCH, 128), pos_table,
                    W.astype(jnp.bfloat16), b.reshape(VOCAB, 1))
    return jnp.transpose(out3, (2, 0, 1))


# speedup vs baseline: 1.0270x; 1.0017x over previous
import functools

import jax
import jax.numpy as jnp
from jax import lax
from jax.experimental import pallas as pl
from jax.experimental.pallas import tpu as pltpu
from jax.experimental.pallas import tpu_sc as plsc

VOCAB = 1000
N_EMBD = 32
T = 8
BATCH = 16384
NROW = BATCH * T
NC = 2
NS = 16
NW = NC * NS
PER_W = NROW // NW
C = 128
NCHUNK = PER_W // C

BBLK = 1024
NBLK = BATCH // BBLK


@functools.cache
def _make_sc_gather():
    mesh = plsc.VectorSubcoreMesh(core_axis_name="c", subcore_axis_name="s")
    return functools.partial(
        pl.kernel,
        out_type=jax.ShapeDtypeStruct((NROW, N_EMBD), jnp.float32),
        mesh=mesh,
        compiler_params=pltpu.CompilerParams(use_tc_tiling_on_sc=False),
        scratch_types=[
            pltpu.VMEM((NCHUNK, C), jnp.int32),
            pltpu.VMEM((NCHUNK, C), jnp.int32),
        ] + [pltpu.VMEM((C, N_EMBD), jnp.float32)] * 4
          + [pltpu.SemaphoreType.DMA] * 8,
    )(_sc_body)


def _sc_body(idx_hbm, tok_hbm, out_hbm, idx_v, sidx_v,
             b0, b1, b2, b3, g0, g1, g2, g3, s0, s1, s2, s3):
    wid = lax.axis_index("s") * NC + lax.axis_index("c")
    base = wid * PER_W
    pltpu.sync_copy(idx_hbm.at[pl.ds(wid * NCHUNK, NCHUNK), :], idx_v)

    iota = lax.iota(jnp.int32, 16)
    t_vec = iota & 7
    soff = (t_vec >> 2) * (4 * BATCH) + (t_vec & 3)

    def pbody(m, carry):
        j0 = base + m * 16
        b_vec = (j0 >> 3) + (iota >> 3)
        k = m // (C // 16)
        col = (m % (C // 16)) * 16
        sidx_v[k, pl.ds(col, 16)] = soff + b_vec * 4
        return carry

    lax.fori_loop(0, PER_W // 16, pbody, 0)

    bufs = (b0, b1, b2, b3)
    gsems = (g0, g1, g2, g3)
    ssems = (s0, s1, s2, s3)

    def g_start(k, p):
        pltpu.async_copy(tok_hbm.at[idx_v.at[k]], bufs[p], gsems[p])

    def g_wait(k, p):
        pltpu.make_async_copy(tok_hbm.at[idx_v.at[k]], bufs[p], gsems[p]).wait()

    def s_start(k, p):
        pltpu.async_copy(bufs[p], out_hbm.at[sidx_v.at[k]], ssems[p])

    def s_wait(k, p):
        pltpu.make_async_copy(bufs[p], out_hbm.at[sidx_v.at[k]], ssems[p]).wait()

    g_start(0, 0)
    g_start(1, 1)

    def lbody(k2, carry):
        for p in range(4):
            k = k2 * 4 + p
            g_wait(k, p)
            s_start(k, p)

            @pl.when(k >= 2)
            def _():
                s_wait(k - 2, (p - 2) % 4)

            @pl.when(k + 2 <= NCHUNK - 1)
            def _():
                g_start(k + 2, (p + 2) % 4)
        return carry

    lax.fori_loop(0, NCHUNK // 4, lbody, 0)
    s_wait(NCHUNK - 2, (NCHUNK - 2) % 4)
    s_wait(NCHUNK - 1, (NCHUNK - 1) % 4)


def _proj_body(g_ref, pos_ref, w_ref, b_ref, out_ref):
    hh = pl.program_id(0)
    for q in range(4):
        x = g_ref[0, :, q * N_EMBD:(q + 1) * N_EMBD] \
            + pos_ref[pl.ds(hh * 4 + q, 1), :]
        y = lax.dot_general(w_ref[...], x.astype(jnp.bfloat16),
                            (((1,), (1,)), ((), ())),
                            preferred_element_type=jnp.float32)
        out_ref[q] = y + b_ref[...]


def _project(g2, pos_table, w_bf, b_col):
    return pl.pallas_call(
        _proj_body,
        grid=(2, NBLK),
        in_specs=[
            pl.BlockSpec((1, BBLK, 128), lambda h, k: (h, k, 0)),
            pl.BlockSpec((T, N_EMBD), lambda h, k: (0, 0)),
            pl.BlockSpec((VOCAB, N_EMBD), lambda h, k: (0, 0)),
            pl.BlockSpec((VOCAB, 1), lambda h, k: (0, 0)),
        ],
        out_specs=pl.BlockSpec((4, VOCAB, BBLK), lambda h, k: (h, 0, k)),
        out_shape=jax.ShapeDtypeStruct((T, VOCAB, BATCH), jnp.float32),
    )(g2, pos_table, w_bf, b_col)


def kernel(idx, tok_table, pos_table, W, b):
    g = _make_sc_gather()(idx.reshape(NROW // C, C), tok_table)
    out3 = _project(g.reshape(2, BATCH, 128), pos_table,
                    W.astype(jnp.bfloat16), b.reshape(VOCAB, 1))
    return jnp.transpose(out3, (2, 0, 1))


# speedup vs baseline: 1.0310x; 1.0039x over previous
import functools

import jax
import jax.numpy as jnp
from jax import lax
from jax.experimental import pallas as pl
from jax.experimental.pallas import tpu as pltpu
from jax.experimental.pallas import tpu_sc as plsc

VOCAB = 1000
N_EMBD = 32
T = 8
BATCH = 16384
NROW = BATCH * T
NC = 2
NS = 16
NW = NC * NS
PER_W = NROW // NW
C = 128
NCHUNK = PER_W // C

BBLK = 512
NBLK = BATCH // BBLK


@functools.cache
def _make_sc_gather():
    mesh = plsc.VectorSubcoreMesh(core_axis_name="c", subcore_axis_name="s")
    return functools.partial(
        pl.kernel,
        out_type=jax.ShapeDtypeStruct((NROW, N_EMBD), jnp.float32),
        mesh=mesh,
        compiler_params=pltpu.CompilerParams(use_tc_tiling_on_sc=False),
        scratch_types=[
            pltpu.VMEM((NCHUNK, C), jnp.int32),
            pltpu.VMEM((NCHUNK, C), jnp.int32),
        ] + [pltpu.VMEM((C, N_EMBD), jnp.float32)] * 4
          + [pltpu.SemaphoreType.DMA] * 8,
    )(_sc_body)


def _sc_body(idx_hbm, tok_hbm, out_hbm, idx_v, sidx_v,
             b0, b1, b2, b3, g0, g1, g2, g3, s0, s1, s2, s3):
    wid = lax.axis_index("s") * NC + lax.axis_index("c")
    base = wid * PER_W
    pltpu.sync_copy(idx_hbm.at[pl.ds(wid * NCHUNK, NCHUNK), :], idx_v)

    iota = lax.iota(jnp.int32, 16)
    t_vec = iota & 7
    soff = (t_vec >> 2) * (4 * BATCH) + (t_vec & 3)

    def pbody(m, carry):
        j0 = base + m * 16
        b_vec = (j0 >> 3) + (iota >> 3)
        k = m // (C // 16)
        col = (m % (C // 16)) * 16
        sidx_v[k, pl.ds(col, 16)] = soff + b_vec * 4
        return carry

    lax.fori_loop(0, PER_W // 16, pbody, 0)

    bufs = (b0, b1, b2, b3)
    gsems = (g0, g1, g2, g3)
    ssems = (s0, s1, s2, s3)

    def g_start(k, p):
        pltpu.async_copy(tok_hbm.at[idx_v.at[k]], bufs[p], gsems[p])

    def g_wait(k, p):
        pltpu.make_async_copy(tok_hbm.at[idx_v.at[k]], bufs[p], gsems[p]).wait()

    def s_start(k, p):
        pltpu.async_copy(bufs[p], out_hbm.at[sidx_v.at[k]], ssems[p])

    def s_wait(k, p):
        pltpu.make_async_copy(bufs[p], out_hbm.at[sidx_v.at[k]], ssems[p]).wait()

    g_start(0, 0)
    g_start(1, 1)

    def lbody(k2, carry):
        for p in range(4):
            k = k2 * 4 + p
            g_wait(k, p)
            s_start(k, p)

            @pl.when(k >= 2)
            def _():
                s_wait(k - 2, (p - 2) % 4)

            @pl.when(k + 2 <= NCHUNK - 1)
            def _():
                g_start(k + 2, (p + 2) % 4)
        return carry

    lax.fori_loop(0, NCHUNK // 4, lbody, 0)
    s_wait(NCHUNK - 2, (NCHUNK - 2) % 4)
    s_wait(NCHUNK - 1, (NCHUNK - 1) % 4)


def _proj_body(g_ref, pos_ref, w_ref, b_ref, out_ref):
    for t in range(T):
        h, q = t // 4, t % 4
        x = g_ref[h, :, q * N_EMBD:(q + 1) * N_EMBD] + pos_ref[pl.ds(t, 1), :]
        y = lax.dot_general(w_ref[...], x.astype(jnp.bfloat16),
                            (((1,), (1,)), ((), ())),
                            preferred_element_type=jnp.float32)
        out_ref[t] = y + b_ref[...]


def _project(g2, pos_table, w_bf, b_col):
    return pl.pallas_call(
        _proj_body,
        grid=(NBLK,),
        in_specs=[
            pl.BlockSpec((2, BBLK, 128), lambda k: (0, k, 0)),
            pl.BlockSpec((T, N_EMBD), lambda k: (0, 0)),
            pl.BlockSpec((VOCAB, N_EMBD), lambda k: (0, 0)),
            pl.BlockSpec((VOCAB, 1), lambda k: (0, 0)),
        ],
        out_specs=pl.BlockSpec((T, VOCAB, BBLK), lambda k: (0, 0, k)),
        out_shape=jax.ShapeDtypeStruct((T, VOCAB, BATCH), jnp.float32),
    )(g2, pos_table, w_bf, b_col)


def kernel(idx, tok_table, pos_table, W, b):
    g = _make_sc_gather()(idx.reshape(NROW // C, C), tok_table)
    out3 = _project(g.reshape(2, BATCH, 128), pos_table,
                    W.astype(jnp.bfloat16), b.reshape(VOCAB, 1))
    return jnp.transpose(out3, (2, 0, 1))
